# baseline (speedup 1.0000x reference)
import functools

import jax
import jax.numpy as jnp
from jax import lax
from jax.experimental import pallas as pl
from jax.experimental.pallas import tpu as pltpu
from jax.experimental.pallas import tpu_sc as plsc

B = 16384
D = 64
NT = 6
OFFSETS = (0, 2, 808, 1743, 2589, 3000)

_info = plsc.get_sparse_core_info()
_NC = _info.num_cores
_NS = _info.num_subcores
NW = _NC * _NS
NOUT = B * NT // NW
NIDX = 128
ROWS_PER_CHUNK = 512
NCHUNKS = NOUT // ROWS_PER_CHUNK
STREAMS_PER_CHUNK = ROWS_PER_CHUNK // NIDX

_mesh = plsc.VectorSubcoreMesh(core_axis_name="c", subcore_axis_name="s")


@functools.partial(
    pl.kernel,
    mesh=_mesh,
    compiler_params=pltpu.CompilerParams(use_tc_tiling_on_sc=False),
    out_type=jax.ShapeDtypeStruct((B * NT, D), jnp.float32),
    scratch_types=[
        pltpu.VMEM((NOUT // NIDX, NIDX), jnp.int32),
        pltpu.VMEM((ROWS_PER_CHUNK, D), jnp.float32),
        pltpu.VMEM((ROWS_PER_CHUNK, D), jnp.float32),
        pltpu.SemaphoreType.DMA,
        pltpu.SemaphoreType.DMA,
    ],
)
def _emb_kernel(cidx_h, wcat_h, out_h, cidx_v, rows_a, rows_b, gsem, wsem):
    wid = lax.axis_index("s") * _NC + lax.axis_index("c")
    nrows = NOUT // NIDX
    pltpu.sync_copy(cidx_h.at[pl.ds(wid * nrows, nrows)], cidx_v)

    bufs = (rows_a, rows_b)
    base = wid * NOUT

    def fire_gathers(c, buf):
        return [
            pltpu.async_copy(
                wcat_h.at[cidx_v.at[c * STREAMS_PER_CHUNK + j]],
                buf.at[pl.ds(j * NIDX, NIDX)], gsem)
            for j in range(STREAMS_PER_CHUNK)
        ]

    writes = [None] * NCHUNKS
    gathers = fire_gathers(0, bufs[0])
    for c in range(NCHUNKS):
        for g in gathers:
            g.wait()
        if c + 1 < NCHUNKS:
            if c >= 1:
                writes[c - 1].wait()
            gathers = fire_gathers(c + 1, bufs[(c + 1) % 2])
        writes[c] = pltpu.async_copy(
            bufs[c % 2],
            out_h.at[pl.ds(base + c * ROWS_PER_CHUNK, ROWS_PER_CHUNK)], wsem)
    writes[NCHUNKS - 2].wait()
    writes[NCHUNKS - 1].wait()


def kernel(cate, customer, brand, campaign, price, pids, id, W_cate,
           W_customer, W_brand, W_campaign, W_price, W_pids, W_id):
    wcat = jnp.concatenate(
        [W_pids, W_cate, W_customer, W_brand, W_campaign, W_price], axis=0)
    offs = jnp.asarray(OFFSETS, dtype=jnp.int32)
    cidx = (jnp.stack([pids, cate, customer, brand, campaign, price], axis=1)
            + offs[None, :]).reshape(B * NT // NIDX, NIDX)
    out = _emb_kernel(cidx, wcat)
    return out.reshape(B, NT, D)
